# baseline (speedup 1.0000x reference)
import jax
import jax.numpy as jnp
from jax.experimental import pallas as pl

_N = 100000
_H = 128
_BLOCK = 25000


def _copy_body(in_ref, out_ref):
    out_ref[...] = in_ref[...]


def kernel(table):
    grid = (_N // _BLOCK,)
    return pl.pallas_call(
        _copy_body,
        grid=grid,
        in_specs=[pl.BlockSpec((_BLOCK, _H), lambda i: (i, 0))],
        out_specs=pl.BlockSpec((_BLOCK, _H), lambda i: (i, 0)),
        out_shape=jax.ShapeDtypeStruct((_N, _H), table.dtype),
    )(table)
